# baseline (speedup 1.0000x reference)
import functools

import jax
import jax.numpy as jnp
from jax import lax
from jax.experimental import pallas as pl
from jax.experimental.pallas import tpu as pltpu
from jax.experimental.pallas import tpu_sc as plsc

_EPS = 1e-06

_NC = 2
_NS = 16
_NW = _NC * _NS


_PB = 256


def _table_body(emb_ref, w_ref, projblk_ref, out_ref, y_ref):
    i = pl.program_id(0)
    nb = pl.num_programs(0)

    @pl.when(i == 0)
    def _():
        emb = emb_ref[...]
        var = jnp.mean(emb * emb, axis=1, keepdims=True)
        y_ref[...] = emb * lax.rsqrt(var + _EPS) * w_ref[...]

    zblk = lax.dot_general(y_ref[...], projblk_ref[...], (((1,), (1,)), ((), ())),
                           preferred_element_type=jnp.float32)
    out_ref[:, pl.ds(i * _PB, _PB)] = zblk

    @pl.when(i == nb - 1)
    def _():
        z = out_ref[...]
        var2 = jnp.mean(z * z, axis=1, keepdims=True)
        out_ref[...] = z * lax.rsqrt(var2 + _EPS)


def _compute_table(embedding_table, hard_norm_weight, proj_weight):
    v, d = embedding_table.shape
    m = proj_weight.shape[0]
    return pl.pallas_call(
        _table_body,
        grid=(m // _PB,),
        in_specs=[
            pl.BlockSpec((v, d), lambda i: (0, 0)),
            pl.BlockSpec((1, d), lambda i: (0, 0)),
            pl.BlockSpec((_PB, d), lambda i: (i, 0)),
        ],
        out_specs=pl.BlockSpec((v, m), lambda i: (0, 0)),
        scratch_shapes=[pltpu.VMEM((v, d), jnp.float32)],
        out_shape=jax.ShapeDtypeStruct((v, m), jnp.float32),
    )(embedding_table, hard_norm_weight.reshape(1, d), proj_weight)


@functools.lru_cache(maxsize=None)
def _make_gather(B, D, V):
    CG = 4
    TG = _NW // CG
    W = D // CG
    TB = B // TG
    CT = 32
    NCH = TB // CT
    L = 16
    mesh = plsc.VectorSubcoreMesh(core_axis_name="c", subcore_axis_name="s")

    @functools.partial(
        pl.kernel,
        mesh=mesh,
        out_type=jax.ShapeDtypeStruct((B, D), jnp.float32),
        scratch_types=[
            pltpu.VMEM((V, W), jnp.float32),
            pltpu.VMEM((TB,), jnp.int32),
            pltpu.VMEM((CT, W), jnp.float32),
            pltpu.VMEM((CT, W), jnp.float32),
            pltpu.SemaphoreType.DMA,
            pltpu.SemaphoreType.DMA,
        ],
        compiler_params=pltpu.CompilerParams(needs_layout_passes=False),
    )
    def gather(table_hbm, idx_hbm, out_hbm, tab_v, idx_v, st_a, st_b, sem_a, sem_b):
        wid = lax.axis_index("s") * _NC + lax.axis_index("c")
        cg = wid % CG
        tg = wid // CG
        pltpu.sync_copy(table_hbm.at[:, pl.ds(cg * W, W)], tab_v)
        pltpu.sync_copy(idx_hbm.at[pl.ds(tg * TB, TB)], idx_v)
        bufs, sems = (st_a, st_b), (sem_a, sem_b)
        lanes = lax.iota(jnp.int32, L)

        def fill_and_send(ch, b):
            stage = bufs[b]

            @plsc.parallel_loop(0, CT)
            def _(t):
                idsplat = plsc.load_gather(
                    idx_v, [jnp.full((L,), ch * CT, jnp.int32) + t]
                )
                for j in range(W // L):
                    v = plsc.load_gather(tab_v, [idsplat, lanes + (j * L)])
                    stage[t, pl.ds(j * L, L)] = v


            pltpu.async_copy(
                stage,
                out_hbm.at[pl.ds(tg * TB + ch * CT, CT), pl.ds(cg * W, W)],
                sems[b],
            )

        def drain(b):
            pltpu.make_async_copy(
                bufs[b], out_hbm.at[pl.ds(tg * TB, CT), pl.ds(cg * W, W)], sems[b]
            ).wait()

        fill_and_send(0, 0)
        fill_and_send(1, 1)

        def body(c2, carry):
            drain(0)
            fill_and_send(c2 * 2, 0)
            drain(1)
            fill_and_send(c2 * 2 + 1, 1)
            return carry

        lax.fori_loop(1, NCH // 2, body, jnp.int32(0))
        drain(0)
        drain(1)

    return gather


def kernel(input_ids, embedding_table, hard_norm_weight, proj_weight):
    table = _compute_table(embedding_table, hard_norm_weight, proj_weight)
    (B,) = input_ids.shape
    D = proj_weight.shape[0]
    return _make_gather(B, D, embedding_table.shape[0])(table, input_ids)


# speedup vs baseline: 1.0272x; 1.0272x over previous
import functools

import jax
import jax.numpy as jnp
from jax import lax
from jax.experimental import pallas as pl
from jax.experimental.pallas import tpu as pltpu
from jax.experimental.pallas import tpu_sc as plsc

_EPS = 1e-06

_NC = 2
_NS = 16
_NW = _NC * _NS


_PB = 512


def _table_body(emb_ref, w_ref, projblk_ref, out_ref, y_ref):
    i = pl.program_id(0)
    nb = pl.num_programs(0)

    @pl.when(i == 0)
    def _():
        emb = emb_ref[...]
        var = jnp.mean(emb * emb, axis=1, keepdims=True)
        y_ref[...] = emb * lax.rsqrt(var + _EPS) * w_ref[...]

    zblk = lax.dot_general(y_ref[...], projblk_ref[...], (((1,), (1,)), ((), ())),
                           preferred_element_type=jnp.float32)
    out_ref[:, pl.ds(i * _PB, _PB)] = zblk

    @pl.when(i == nb - 1)
    def _():
        z = out_ref[...]
        var2 = jnp.mean(z * z, axis=1, keepdims=True)
        out_ref[...] = z * lax.rsqrt(var2 + _EPS)


def _compute_table(embedding_table, hard_norm_weight, proj_weight):
    v, d = embedding_table.shape
    m = proj_weight.shape[0]
    return pl.pallas_call(
        _table_body,
        grid=(m // _PB,),
        in_specs=[
            pl.BlockSpec((v, d), lambda i: (0, 0)),
            pl.BlockSpec((1, d), lambda i: (0, 0)),
            pl.BlockSpec((_PB, d), lambda i: (i, 0)),
        ],
        out_specs=pl.BlockSpec((v, m), lambda i: (0, 0)),
        scratch_shapes=[pltpu.VMEM((v, d), jnp.float32)],
        out_shape=jax.ShapeDtypeStruct((v, m), jnp.float32),
    )(embedding_table, hard_norm_weight.reshape(1, d), proj_weight)


@functools.lru_cache(maxsize=None)
def _make_gather(B, D, V):
    CG = 4
    TG = _NW // CG
    W = D // CG
    TB = B // TG
    CT = 32
    NCH = TB // CT
    L = 16
    mesh = plsc.VectorSubcoreMesh(core_axis_name="c", subcore_axis_name="s")

    @functools.partial(
        pl.kernel,
        mesh=mesh,
        out_type=jax.ShapeDtypeStruct((B, D), jnp.float32),
        scratch_types=[
            pltpu.VMEM((V, W), jnp.float32),
            pltpu.VMEM((TB,), jnp.int32),
            pltpu.VMEM((CT, W), jnp.float32),
            pltpu.VMEM((CT, W), jnp.float32),
            pltpu.SemaphoreType.DMA,
            pltpu.SemaphoreType.DMA,
        ],
        compiler_params=pltpu.CompilerParams(needs_layout_passes=False),
    )
    def gather(table_hbm, idx_hbm, out_hbm, tab_v, idx_v, st_a, st_b, sem_a, sem_b):
        wid = lax.axis_index("s") * _NC + lax.axis_index("c")
        cg = wid % CG
        tg = wid // CG
        pltpu.sync_copy(table_hbm.at[:, pl.ds(cg * W, W)], tab_v)
        pltpu.sync_copy(idx_hbm.at[pl.ds(tg * TB, TB)], idx_v)
        bufs, sems = (st_a, st_b), (sem_a, sem_b)
        lanes = lax.iota(jnp.int32, L)

        def fill_and_send(ch, b):
            stage = bufs[b]

            @plsc.parallel_loop(0, CT)
            def _(t):
                idsplat = plsc.load_gather(
                    idx_v, [jnp.full((L,), ch * CT, jnp.int32) + t]
                )
                for j in range(W // L):
                    v = plsc.load_gather(tab_v, [idsplat, lanes + (j * L)])
                    stage[t, pl.ds(j * L, L)] = v


            pltpu.async_copy(
                stage,
                out_hbm.at[pl.ds(tg * TB + ch * CT, CT), pl.ds(cg * W, W)],
                sems[b],
            )

        def drain(b):
            pltpu.make_async_copy(
                bufs[b], out_hbm.at[pl.ds(tg * TB, CT), pl.ds(cg * W, W)], sems[b]
            ).wait()

        fill_and_send(0, 0)
        fill_and_send(1, 1)

        def body(c2, carry):
            drain(0)
            fill_and_send(c2 * 2, 0)
            drain(1)
            fill_and_send(c2 * 2 + 1, 1)
            return carry

        lax.fori_loop(1, NCH // 2, body, jnp.int32(0))
        drain(0)
        drain(1)

    return gather


def kernel(input_ids, embedding_table, hard_norm_weight, proj_weight):
    table = _compute_table(embedding_table, hard_norm_weight, proj_weight)
    (B,) = input_ids.shape
    D = proj_weight.shape[0]
    return _make_gather(B, D, embedding_table.shape[0])(table, input_ids)
